# baseline (speedup 1.0000x reference)
import functools

import jax
import jax.numpy as jnp
import numpy as np
from jax.experimental import pallas as pl
from jax.experimental.pallas import tpu as pltpu
from jax.sharding import Mesh, PartitionSpec as P

N = 256
G = 4
NG = N // G
RBLK = 64
NSTEP = N // RBLK
EW = G * 32
OHW = RBLK + NG


def _dot(a, b, out=jnp.float32):
    return jax.lax.dot_general(
        a, b, (((1,), (0,)), ((), ())), preferred_element_type=out
    )


def _body(layer_meta, *refs):
    nf_ref, maskc_ref, maskjg_ref, s_ref, f32_ref, f4_ref = refs[:6]
    wrefs = refs[6:-3]
    nf_out_ref, ef_out_ref, e_ref = refs[-3:]
    bf16 = jnp.bfloat16

    @pl.when(pl.program_id(0) == 0)
    def _init():
        row = jax.lax.broadcasted_iota(jnp.int32, (N * NG, OHW), 0)
        col = jax.lax.broadcasted_iota(jnp.int32, (N * NG, OHW), 1)
        ii = (row // NG) % RBLK
        gg = row % NG
        oh = jnp.where(
            (col == ii) | (col == RBLK + gg), 1.0, 0.0
        ).astype(bf16)
        e_ref[:, EW:EW + OHW] = oh

    nf = nf_ref[0]
    mc = maskc_ref[0]
    mjg = maskjg_ref[0]
    smat = s_ref[...]
    f32 = f32_ref[...]
    f4 = f4_ref[...]

    widx = 0
    n_layers = len(layer_meta)
    for l, (nd, ed, eo, no) in enumerate(layer_meta):
        first = l == 0
        last = l == n_layers - 1
        w1i = wrefs[widx][...]; widx += 1
        b1 = wrefs[widx][...]; widx += 1
        w1j = wrefs[widx][...]; widx += 1
        if not first:
            w1e_bd = wrefs[widx][...]; widx += 1
        w2_bd = wrefs[widx][...]; widx += 1
        b2r = wrefs[widx][...]; widx += 1
        w1na = wrefs[widx][...]; widx += 1
        w1nb = wrefs[widx][...]; widx += 1
        b1n = wrefs[widx][...]; widx += 1
        w2n = wrefs[widx][...]; widx += 1
        b2n = wrefs[widx][...]; widx += 1

        a_full = _dot(nf, w1i) + b1
        b_full = _dot(nf, w1j)
        sb = _dot(smat, b_full)
        bg = jnp.concatenate(
            [sb[0:NG], sb[NG:2 * NG], sb[2 * NG:3 * NG], sb[3 * NG:4 * NG]],
            axis=1,
        ).astype(bf16)

        fold = f4 if last else f32
        aggs = []
        for r in range(NSTEP):
            rows = pl.ds(r * RBLK * NG, RBLK * NG)
            a_blk = a_full[r * RBLK:(r + 1) * RBLK]
            a_rep = jnp.concatenate([a_blk] * G, axis=1).astype(bf16)
            if first:
                e_oh = e_ref[rows, EW:EW + OHW]
                w_aug = jnp.concatenate([a_rep, bg], axis=0)
                pre = _dot(e_oh, w_aug)
            else:
                e_full = e_ref[rows, :]
                w_aug = jnp.concatenate([w1e_bd, a_rep, bg], axis=0)
                pre = _dot(e_full, w_aug)
            h = jnp.maximum(pre, 0.0).astype(bf16)
            enew = _dot(h, w2_bd) + b2r
            en3 = enew.reshape(RBLK, NG, G * eo)
            s1 = jnp.sum(en3, axis=1)
            aggs.append(_dot(s1, fold) * (1.0 / N))
            if last:
                out = en3 * mc[r * RBLK:(r + 1) * RBLK][:, :, None]
                out = out * mjg[None, :, :]
                out4 = out.reshape(RBLK, 8, 8, G * eo)
                ri = pl.ds(r * RBLK, RBLK)
                for mm in range(8):
                    ef_out_ref[0, ri, :, pl.ds(mm * G * eo, G * eo)] = (
                        out4[:, :, mm, :]
                    )
            elif first:
                e_ref[rows, 0:EW] = enew.astype(bf16)
            else:
                e_ref[rows, 0:EW] = (
                    e_full[:, 0:EW].astype(jnp.float32) + enew
                ).astype(bf16)
        agg = jnp.concatenate(aggs, axis=0)

        hn = jnp.maximum(_dot(nf, w1na) + _dot(agg, w1nb) + b1n, 0.0)
        node_out = _dot(hn, w2n) + b2n
        if first:
            nf = node_out
        elif last:
            nf_out_ref[0] = node_out * mc
        else:
            nf = nf + node_out


def kernel(node_feat, mask, params):
    bsz = node_feat.shape[0]
    f32t = jnp.float32
    bf16 = jnp.bfloat16
    eye4 = jnp.eye(G, dtype=f32t)

    layer_meta = []
    flat = []
    specs = []

    def add(arr):
        flat.append(arr)
        specs.append(
            pl.BlockSpec(arr.shape, lambda b: (0,) * arr.ndim)
        )

    eo_prev = 0
    for l, p in enumerate(params):
        we, wn = p["edge"], p["node"]
        w1, b1, w2, b2 = we["W1"], we["b1"], we["W2"], we["b2"]
        ed = 0 if l == 0 else eo_prev
        nd = (w1.shape[0] - ed) // 2
        eo = w2.shape[1]
        no = wn["W2"].shape[1]
        layer_meta.append((nd, ed, eo, no))
        eo_prev = eo

        add(w1[:nd])
        add(b1[None, :])
        add(w1[nd:2 * nd])
        if l > 0:
            add(jnp.kron(eye4, w1[2 * nd:]).astype(bf16))
        add(jnp.kron(eye4, w2).astype(bf16))
        add(jnp.tile(b2, G)[None, :])
        w1n = wn["W1"]
        add(w1n[:nd])
        add(w1n[nd:])
        add(wn["b1"][None, :])
        add(wn["W2"])
        add(wn["b2"][None, :])

    ridx = jnp.arange(N)
    smat = jnp.zeros((N, N), f32t).at[ridx, G * (ridx % NG) + ridx // NG].set(1.0)
    fold32 = jnp.tile(jnp.eye(32, dtype=f32t), (G, 1))
    fold4 = jnp.tile(jnp.eye(4, dtype=f32t), (G, 1))

    maskc = mask[:, :, None]
    mjg = jnp.repeat(mask.reshape(bsz, NG, G), 4, axis=2)

    eo_last = layer_meta[-1][2]
    no_last = layer_meta[-1][3]

    in_specs = [
        pl.BlockSpec((1, N, node_feat.shape[-1]), lambda b: (b, 0, 0)),
        pl.BlockSpec((1, N, 1), lambda b: (b, 0, 0)),
        pl.BlockSpec((1, NG, G * eo_last), lambda b: (b, 0, 0)),
        pl.BlockSpec((N, N), lambda b: (0, 0)),
        pl.BlockSpec((G * 32, 32), lambda b: (0, 0)),
        pl.BlockSpec((G * 4, 4), lambda b: (0, 0)),
    ] + specs

    out_specs = [
        pl.BlockSpec((1, N, no_last), lambda b: (b, 0, 0)),
        pl.BlockSpec((1, N, 8, 128), lambda b: (b, 0, 0, 0)),
    ]

    def run(nf_in, mc_in, mjg_in, smat_in, f32_in, f4_in, *w):
        lb = nf_in.shape[0]
        return pl.pallas_call(
            functools.partial(_body, layer_meta),
            grid=(lb,),
            in_specs=in_specs,
            out_specs=out_specs,
            out_shape=[
                jax.ShapeDtypeStruct((lb, N, no_last), f32t),
                jax.ShapeDtypeStruct((lb, N, 8, 128), f32t),
            ],
            scratch_shapes=[pltpu.VMEM((N * NG, EW + OHW), bf16)],
        )(nf_in, mc_in, mjg_in, smat_in, f32_in, f4_in, *w)

    ndev = 1
    try:
        devs = jax.devices()
        for cand in (4, 2):
            if len(devs) >= cand and bsz % cand == 0:
                ndev = cand
                break
    except Exception:
        ndev = 1

    args = (node_feat, maskc, mjg, smat, fold32, fold4) + tuple(flat)
    if ndev > 1:
        mesh = Mesh(np.asarray(devs[:ndev]), ("d",))
        sharded = (P("d"), P("d"), P("d"))
        repl = (P(),) * (3 + len(flat))
        run_sharded = jax.shard_map(
            run, mesh=mesh,
            in_specs=sharded + repl,
            out_specs=(P("d"), P("d")),
            check_vma=False,
        )
        nf_out, ef_out = run_sharded(*args)
    else:
        nf_out, ef_out = run(*args)

    return nf_out, ef_out.reshape(bsz, N, N, eo_last)


# speedup vs baseline: 1.6987x; 1.6987x over previous
import functools

import jax
import jax.numpy as jnp
from jax.experimental import pallas as pl
from jax.experimental.pallas import tpu as pltpu

N = 256
G = 4
NG = N // G
RBLK = 64
NSTEP = N // RBLK
EW = G * 32
OHW = RBLK + NG


def _dot(a, b, out=jnp.float32):
    return jax.lax.dot_general(
        a, b, (((1,), (0,)), ((), ())), preferred_element_type=out
    )


def _body(layer_meta, *refs):
    nf_ref, maskc_ref, maskjg_ref, s_ref, f4_ref = refs[:5]
    wrefs = refs[5:-4]
    nf_out_ref, ef_out_ref, e_ref, w_ref = refs[-4:]
    bf16 = jnp.bfloat16

    @pl.when(pl.program_id(0) == 0)
    def _init():
        row = jax.lax.broadcasted_iota(jnp.int32, (N * NG, OHW), 0)
        col = jax.lax.broadcasted_iota(jnp.int32, (N * NG, OHW), 1)
        ii = (row // NG) % RBLK
        gg = row % NG
        oh = jnp.where(
            (col == ii) | (col == RBLK + gg), 1.0, 0.0
        ).astype(bf16)
        e_ref[:, EW:EW + OHW] = oh

    nf = nf_ref[0]
    mc = maskc_ref[0]
    mjg = maskjg_ref[0]
    smat = s_ref[...]
    f4 = f4_ref[...]

    widx = 0
    n_layers = len(layer_meta)
    for l, (nd, ed, eo, no) in enumerate(layer_meta):
        first = l == 0
        last = l == n_layers - 1
        w1i = wrefs[widx][...]; widx += 1
        b1 = wrefs[widx][...]; widx += 1
        w1j = wrefs[widx][...]; widx += 1
        if not first:
            w_ref[0:EW, :] = wrefs[widx][...]
            widx += 1
        w2x = wrefs[widx][...]; widx += 1
        b2r = wrefs[widx][...]; widx += 1
        if not last:
            b2row = wrefs[widx][...]; widx += 1
        w1na = wrefs[widx][...]; widx += 1
        w1nb = wrefs[widx][...]; widx += 1
        b1n = wrefs[widx][...]; widx += 1
        w2n = wrefs[widx][...]; widx += 1
        b2n = wrefs[widx][...]; widx += 1

        a_full = _dot(nf, w1i) + b1
        b_full = _dot(nf, w1j)
        sb = _dot(smat, b_full)
        bg = jnp.concatenate(
            [sb[0:NG], sb[NG:2 * NG], sb[2 * NG:3 * NG], sb[3 * NG:4 * NG]],
            axis=1,
        ).astype(bf16)
        w_ref[EW + RBLK:, :] = bg

        aggs = []
        for r in range(NSTEP):
            rows = pl.ds(r * RBLK * NG, RBLK * NG)
            a_blk = a_full[r * RBLK:(r + 1) * RBLK]
            w_ref[EW:EW + RBLK, :] = (
                jnp.concatenate([a_blk] * G, axis=1).astype(bf16)
            )
            if first:
                pre = _dot(e_ref[rows, EW:], w_ref[EW:, :])
            else:
                e_full = e_ref[rows, :]
                pre = _dot(e_full, w_ref[...])
            h = jnp.maximum(pre.astype(bf16), bf16(0))
            if last:
                enew = _dot(h, w2x) + b2r
                en3 = enew.reshape(RBLK, NG, G * eo)
                s1 = jnp.sum(en3, axis=1)
                aggs.append(_dot(s1, f4) * (1.0 / N))
                out = en3 * mc[r * RBLK:(r + 1) * RBLK][:, :, None]
                out = out * mjg[None, :, :]
                out4 = out.reshape(RBLK, 8, 8, G * eo)
                ri = pl.ds(r * RBLK, RBLK)
                for mm in range(8):
                    ef_out_ref[0, ri, :, pl.ds(mm * G * eo, G * eo)] = (
                        out4[:, :, mm, :]
                    )
            else:
                ex = _dot(h, w2x)
                enew = ex[:, 0:G * eo] + b2r
                ap3 = ex[:, G * eo:].reshape(RBLK, NG, eo)
                s1 = jnp.sum(ap3, axis=1)
                aggs.append(s1 * (1.0 / N) + b2row)
                if first:
                    e_ref[rows, 0:EW] = enew.astype(bf16)
                else:
                    e_ref[rows, 0:EW] = (
                        e_full[:, 0:EW].astype(jnp.float32) + enew
                    ).astype(bf16)
        agg = jnp.concatenate(aggs, axis=0)

        hn = jnp.maximum(_dot(nf, w1na) + _dot(agg, w1nb) + b1n, 0.0)
        node_out = _dot(hn, w2n) + b2n
        if first:
            nf = node_out
        elif last:
            nf_out_ref[0] = node_out * mc
        else:
            nf = nf + node_out


def kernel(node_feat, mask, params):
    bsz = node_feat.shape[0]
    f32t = jnp.float32
    bf16 = jnp.bfloat16
    eye4 = jnp.eye(G, dtype=f32t)

    layer_meta = []
    flat = []
    specs = []

    def add(arr):
        flat.append(arr)
        specs.append(
            pl.BlockSpec(arr.shape, lambda b: (0,) * arr.ndim)
        )

    eo_prev = 0
    for l, p in enumerate(params):
        we, wn = p["edge"], p["node"]
        w1, b1, w2, b2 = we["W1"], we["b1"], we["W2"], we["b2"]
        ed = 0 if l == 0 else eo_prev
        nd = (w1.shape[0] - ed) // 2
        eo = w2.shape[1]
        no = wn["W2"].shape[1]
        layer_meta.append((nd, ed, eo, no))
        eo_prev = eo

        add(w1[:nd])
        add(b1[None, :])
        add(w1[nd:2 * nd])
        if l > 0:
            add(jnp.kron(eye4, w1[2 * nd:]).astype(bf16))
        w2bd = jnp.kron(eye4, w2)
        if l < len(params) - 1:
            w2bd = jnp.concatenate([w2bd, jnp.tile(w2, (G, 1))], axis=1)
        add(w2bd.astype(bf16))
        add(jnp.tile(b2, G)[None, :])
        if l < len(params) - 1:
            add(b2[None, :])
        w1n = wn["W1"]
        add(w1n[:nd])
        add(w1n[nd:])
        add(wn["b1"][None, :])
        add(wn["W2"])
        add(wn["b2"][None, :])

    ridx = jnp.arange(N)
    smat = jnp.zeros((N, N), f32t).at[ridx, G * (ridx % NG) + ridx // NG].set(1.0)
    fold4 = jnp.tile(jnp.eye(4, dtype=f32t), (G, 1))

    maskc = mask[:, :, None]
    mjg = jnp.repeat(mask.reshape(bsz, NG, G), 4, axis=2)

    eo_last = layer_meta[-1][2]
    no_last = layer_meta[-1][3]

    in_specs = [
        pl.BlockSpec((1, N, node_feat.shape[-1]), lambda b: (b, 0, 0)),
        pl.BlockSpec((1, N, 1), lambda b: (b, 0, 0)),
        pl.BlockSpec((1, NG, G * eo_last), lambda b: (b, 0, 0)),
        pl.BlockSpec((N, N), lambda b: (0, 0)),
        pl.BlockSpec((G * 4, 4), lambda b: (0, 0)),
    ] + specs

    out_specs = [
        pl.BlockSpec((1, N, no_last), lambda b: (b, 0, 0)),
        pl.BlockSpec((1, N, 8, 128), lambda b: (b, 0, 0, 0)),
    ]

    def run(nf_in, mc_in, mjg_in, smat_in, f4_in, *w):
        lb = nf_in.shape[0]
        return pl.pallas_call(
            functools.partial(_body, layer_meta),
            grid=(lb,),
            in_specs=in_specs,
            out_specs=out_specs,
            out_shape=[
                jax.ShapeDtypeStruct((lb, N, no_last), f32t),
                jax.ShapeDtypeStruct((lb, N, 8, 128), f32t),
            ],
            scratch_shapes=[
                pltpu.VMEM((N * NG, EW + OHW), bf16),
                pltpu.VMEM((EW + OHW, 256), bf16),
            ],
        )(nf_in, mc_in, mjg_in, smat_in, f4_in, *w)

    args = (node_feat, maskc, mjg, smat, fold4) + tuple(flat)
    nf_out, ef_out = run(*args)

    return nf_out, ef_out.reshape(bsz, N, N, eo_last)
